# Initial kernel scaffold; baseline (speedup 1.0000x reference)
import jax
import jax.numpy as jnp
from jax.experimental import pallas as pl


def kernel(ent_emb, rel_emb, nei_array):
    raise NotImplementedError("write your pallas kernel here")


# speedup vs baseline: 4.8085x; 4.8085x over previous
import functools

import jax
import jax.numpy as jnp
from jax import lax
from jax.experimental import pallas as pl
from jax.experimental.pallas import tpu as pltpu
from jax.experimental.pallas import tpu_sc as plsc

_L = 16
_NS = 16
_C = 128


def _pad_rows(n: int) -> int:
  blk = _NS * _C
  return -(-n // blk) * blk


def _build_sc_kernel(np_e: int, np_r: int, n_edges: int, half: int):
  n_chunks = n_edges // _C
  chunks_per_tile = -(-n_chunks // _NS)
  rows_e = np_e // _NS
  rows_r = np_r // _NS
  assert n_edges % _C == 0 and half % _L == 0

  mesh = plsc.VectorSubcoreMesh(core_axis_name="c", subcore_axis_name="s")

  @functools.partial(
      pl.kernel,
      out_type=(
          jax.ShapeDtypeStruct((2 * np_e, half), jnp.float32),
          jax.ShapeDtypeStruct((2 * np_r, half), jnp.float32),
      ),
      mesh=mesh,
      compiler_params=pltpu.CompilerParams(use_tc_tiling_on_sc=False),
      scratch_types=[
          pltpu.VMEM_SHARED((np_e, half), jnp.float32),
          pltpu.VMEM_SHARED((np_r, half), jnp.float32),
          pltpu.VMEM((_C,), jnp.int32),
          pltpu.VMEM((_C,), jnp.int32),
          pltpu.VMEM((_C,), jnp.int32),
          pltpu.VMEM((_C,), jnp.int32),
          pltpu.VMEM((_C,), jnp.int32),
          pltpu.VMEM((_C,), jnp.int32),
          pltpu.VMEM((_C, half), jnp.float32),
          pltpu.VMEM((_C, half), jnp.float32),
          pltpu.VMEM((_C, half), jnp.float32),
          pltpu.VMEM((_C, half), jnp.float32),
          pltpu.VMEM((_C, half), jnp.float32),
          pltpu.SemaphoreType.DMA,
      ],
  )
  def sc_kernel(e2, r2, hh, rr, tt, oe, out_r, acc_e, acc_r,
                ih, ir, it, ioh, ior, iot, gh, gr, gt, vb, mb, sem):
    c = lax.axis_index("c")
    s = lax.axis_index("s")

    pltpu.sync_copy(e2.at[pl.ds(c * np_e + s * rows_e, rows_e)],
                    acc_e.at[pl.ds(s * rows_e, rows_e)])
    pltpu.sync_copy(r2.at[pl.ds(c * np_r + s * rows_r, rows_r)],
                    acc_r.at[pl.ds(s * rows_r, rows_r)])
    plsc.subcore_barrier()

    coff_e = c * np_e
    coff_r = c * np_r

    @pl.loop(0, chunks_per_tile)
    def _chunks(g):
      cid = g * _NS + s

      @pl.when(cid < n_chunks)
      def _():
        base = cid * _C
        pltpu.sync_copy(hh.at[pl.ds(base, _C)], ih)
        pltpu.sync_copy(rr.at[pl.ds(base, _C)], ir)
        pltpu.sync_copy(tt.at[pl.ds(base, _C)], it)
        for k in range(_C // _L):
          sl = pl.ds(k * _L, _L)
          ioh[sl] = ih[sl] + coff_e
          ior[sl] = ir[sl] + coff_r
          iot[sl] = it[sl] + coff_e
        d1 = pltpu.async_copy(e2.at[ioh], gh, sem)
        d2 = pltpu.async_copy(r2.at[ior], gr, sem)
        d3 = pltpu.async_copy(e2.at[iot], gt, sem)
        d1.wait()
        d2.wait()
        d3.wait()

        @pl.loop(0, _C)
        def _rows(row):
          for k in range(half // _L):
            sl = pl.ds(k * _L, _L)
            v = gh[row, sl] + gr[row, sl] - gt[row, sl]
            vb[row, sl] = v
            mb[row, sl] = -v

        pltpu.sync_copy(mb, acc_e.at[ih], add=True)
        pltpu.sync_copy(vb, acc_e.at[it], add=True)
        pltpu.sync_copy(mb, acc_r.at[ir], add=True)

    plsc.subcore_barrier()

    def relu_out(acc, out_ref, coff, rows):
      for b in range(rows // _C):
        row0 = s * rows + b * _C
        pltpu.sync_copy(acc.at[pl.ds(row0, _C)], gh)

        @pl.loop(0, _C)
        def _rl(row):
          for k in range(half // _L):
            sl = pl.ds(k * _L, _L)
            gh[row, sl] = jnp.maximum(gh[row, sl], 0.0)

        pltpu.sync_copy(gh, out_ref.at[pl.ds(coff + row0, _C)])

    relu_out(acc_e, oe, coff_e, rows_e)
    relu_out(acc_r, out_r, coff_r, rows_r)

  return sc_kernel


def kernel(ent_emb, rel_emb, nei_array):
  n_nodes, d = ent_emb.shape
  n_rels = rel_emb.shape[0]
  n_edges = nei_array.shape[0]
  half = d // 2
  np_e = _pad_rows(n_nodes)
  np_r = _pad_rows(n_rels)

  nei = nei_array.astype(jnp.int32)
  h_idx = nei[:, 0]
  r_idx = nei[:, 1]
  t_idx = nei[:, 2]

  def stack(tab, np_n):
    n = tab.shape[0]
    pad = jnp.zeros((np_n - n, half), jnp.float32)
    return jnp.concatenate([tab[:, :half], pad, tab[:, half:], pad], axis=0)

  e2 = stack(ent_emb, np_e)
  r2 = stack(rel_emb, np_r)

  oe2, or2 = _build_sc_kernel(np_e, np_r, n_edges, half)(
      e2, r2, h_idx, r_idx, t_idx)

  ent_out = jnp.concatenate([oe2[:n_nodes], oe2[np_e:np_e + n_nodes]], axis=1)
  rel_out = jnp.concatenate([or2[:n_rels], or2[np_r:np_r + n_rels]], axis=1)
  return ent_out, rel_out
